# baseline (speedup 1.0000x reference)
import functools

import jax
import jax.numpy as jnp
from jax import lax
from jax.experimental import pallas as pl
from jax.experimental.pallas import tpu as pltpu
from jax.experimental.pallas import tpu_sc as plsc

_TC_LANES = 128
_SC_LANES = 16

_VT_SPLIT = 80032
_SC_SHARDS = 8
_SC_TILE_ROWS = 8
_SC_CHUNK_TILES = 8


def _argmax_tc(xT, vt_rows):
    V, R = xT.shape
    C = 10240
    nb = (vt_rows + C - 1) // C
    big = jnp.iinfo(jnp.int32).max

    def body(x_ref, om_ref, oi_ref, vm_ref, vt_ref):
        j = pl.program_id(0)

        @pl.when(j == 0)
        def _init():
            vm_ref[...] = jnp.full((1, R), -jnp.inf, jnp.float32)
            vt_ref[...] = jnp.zeros((1, R), jnp.int32)

        def scan(valid_rows):
            blk = x_ref[...]
            iota0 = lax.broadcasted_iota(jnp.int32, (C, R), 0)
            if valid_rows is not None:
                blk = jnp.where(iota0 < valid_rows, blk, -jnp.inf)
            m = jnp.max(blk, axis=0, keepdims=True)
            im = jnp.min(jnp.where(blk == m, iota0, big), axis=0,
                         keepdims=True) + j * C
            upd = m > vm_ref[...]
            vm_ref[...] = jnp.where(upd, m, vm_ref[...])
            vt_ref[...] = jnp.where(upd, im, vt_ref[...])

        tail = vt_rows - (nb - 1) * C

        @pl.when(j < nb - 1)
        def _steady():
            scan(None)

        @pl.when(j == nb - 1)
        def _last():
            scan(None if tail == C else tail)
            om_ref[...] = vm_ref[...].reshape(R)
            oi_ref[...] = vt_ref[...].reshape(R)

    return pl.pallas_call(
        body,
        grid=(nb,),
        in_specs=[pl.BlockSpec((C, R), lambda j: (j, 0))],
        out_specs=[pl.BlockSpec((R,), lambda j: (0,)),
                   pl.BlockSpec((R,), lambda j: (0,))],
        out_shape=[jax.ShapeDtypeStruct((R,), jnp.float32),
                   jax.ShapeDtypeStruct((R,), jnp.int32)],
        scratch_shapes=[
            pltpu.VMEM((1, R), jnp.float32),
            pltpu.VMEM((1, R), jnp.int32),
        ],
    )(xT)


def _argmax_sc_partial(xT):
    V, R = xT.shape
    L = _SC_LANES
    vs = V - _VT_SPLIT
    rows_per_shard = vs // _SC_SHARDS
    chunk_rows = _SC_TILE_ROWS * _SC_CHUNK_TILES
    n_chunks = rows_per_shard // chunk_rows
    assert rows_per_shard % chunk_rows == 0 and _VT_SPLIT % 8 == 0
    n_tok_blocks = R // _TC_LANES
    vpr = _TC_LANES // L
    mesh = plsc.VectorSubcoreMesh(core_axis_name="c", subcore_axis_name="s",
                                  num_cores=2, num_subcores=16)

    @functools.partial(
        pl.kernel,
        out_type=(jax.ShapeDtypeStruct((_SC_SHARDS * R,), jnp.float32),
                  jax.ShapeDtypeStruct((_SC_SHARDS * R,), jnp.int32)),
        mesh=mesh,
        compiler_params=pltpu.CompilerParams(needs_layout_passes=False,
                                             use_tc_tiling_on_sc=True),
        scratch_types=[
            pltpu.VMEM((chunk_rows, _TC_LANES), jnp.float32),
            pltpu.VMEM((_TC_LANES,), jnp.float32),
            pltpu.VMEM((_TC_LANES,), jnp.int32),
        ],
    )
    def k(x_hbm, pm_hbm, pi_hbm, buf, om_v, oi_v):
        wid = lax.axis_index("s") * 2 + lax.axis_index("c")
        q = wid // n_tok_blocks
        jb = wid % n_tok_blocks
        row0 = _VT_SPLIT + q * rows_per_shard
        col0 = jb * _TC_LANES

        neg_inf = jnp.full((L,), -jnp.inf, jnp.float32)
        zeros = jnp.zeros((L,), jnp.int32)
        init = tuple([neg_inf] * vpr + [zeros] * vpr)

        def step(c, carry):
            rbase = row0 + c * chunk_rows
            pltpu.sync_copy(
                x_hbm.at[pl.ds(rbase, chunk_rows), pl.ds(col0, _TC_LANES)],
                buf)
            acc = list(carry)
            for rr in range(chunk_rows):
                rid = rbase + rr
                for kk in range(vpr):
                    v = buf[rr, pl.ds(kk * L, L)]
                    pred = v > acc[kk]
                    acc[kk] = jnp.where(pred, v, acc[kk])
                    acc[vpr + kk] = jnp.where(pred, rid, acc[vpr + kk])
            return tuple(acc)

        acc = lax.fori_loop(0, n_chunks, step, init, unroll=False)
        for kk in range(vpr):
            om_v[pl.ds(kk * L, L)] = acc[kk]
            oi_v[pl.ds(kk * L, L)] = acc[vpr + kk]
        pltpu.sync_copy(om_v, pm_hbm.at[pl.ds(q * R + col0, _TC_LANES)])
        pltpu.sync_copy(oi_v, pi_hbm.at[pl.ds(q * R + col0, _TC_LANES)])

    return k(xT)


def _reject_sc(draft, tc_max, tc_idx, pm, pi, cu, bonus, params):
    B = cu.shape[0]
    T = draft.shape[0]
    S = T // B
    L = _SC_LANES
    NW = B // L
    W = S + 1
    R = tc_max.shape[0]
    mesh = plsc.VectorSubcoreMesh(core_axis_name="c", subcore_axis_name="s",
                                  num_cores=2, num_subcores=16)

    @functools.partial(
        pl.kernel,
        out_type=(jax.ShapeDtypeStruct((B, W), jnp.int32),
                  jax.ShapeDtypeStruct((B,), jnp.int32)),
        mesh=mesh,
        compiler_params=pltpu.CompilerParams(needs_layout_passes=False),
        scratch_types=[
            pltpu.VMEM((B,), jnp.int32),
            pltpu.VMEM((T,), jnp.int32),
            pltpu.VMEM((R,), jnp.float32),
            pltpu.VMEM((R,), jnp.int32),
            pltpu.VMEM((_SC_SHARDS * R,), jnp.float32),
            pltpu.VMEM((_SC_SHARDS * R,), jnp.int32),
            pltpu.VMEM((R,), jnp.int32),
            pltpu.VMEM((L,), jnp.int32),
            pltpu.VMEM((L,), jnp.int32),
            pltpu.VMEM((L, W), jnp.int32),
            pltpu.VMEM((L,), jnp.int32),
        ],
    )
    def k(draft_hbm, tcm_hbm, tci_hbm, pm_hbm, pi_hbm, cu_hbm, bonus_hbm,
          params_hbm, out_hbm, nb_hbm,
          cu_v, draft_v, tcm_v, tci_v, pm_v, pi_v, tmax_v, bonus_v, params_v,
          out_local, nb_local):
        wid = lax.axis_index("s") * 2 + lax.axis_index("c")

        @pl.when(wid < NW)
        def _():
            pltpu.sync_copy(cu_hbm, cu_v)
            pltpu.sync_copy(draft_hbm, draft_v)
            pltpu.sync_copy(tcm_hbm, tcm_v)
            pltpu.sync_copy(tci_hbm, tci_v)
            pltpu.sync_copy(pm_hbm, pm_v)
            pltpu.sync_copy(pi_hbm, pi_v)
            pltpu.sync_copy(bonus_hbm.at[pl.ds(wid * L, L)], bonus_v)
            pltpu.sync_copy(params_hbm, params_v)

            for kk in range(R // L):
                sl = pl.ds(kk * L, L)
                m = tcm_v[sl]
                idx = tci_v[sl]
                for q in range(_SC_SHARDS):
                    qs = pl.ds(q * R + kk * L, L)
                    pmq = pm_v[qs]
                    pred = pmq > m
                    m = jnp.where(pred, pmq, m)
                    idx = jnp.where(pred, pi_v[qs], idx)
                tmax_v[sl] = idx

            lanes = lax.broadcasted_iota(jnp.int32, (L,), 0)
            r = wid * L + lanes
            cu_r = plsc.load_gather(cu_v, [r])
            cu_rm1 = plsc.load_gather(cu_v, [jnp.maximum(r - 1, 0)])
            zero = jnp.zeros((L,), jnp.int32)
            starts = jnp.where(r == 0, zero, cu_rm1)
            lengths = cu_r - starts
            pos_off = params_v[...]

            prefix = jnp.ones((L,), jnp.int32)
            nm = zero
            for p in range(S):
                pos = pos_off + p
                idx = starts + pos
                valid = pos < lengths
                idx_c = jnp.clip(idx, 0, T - 1)
                dr = plsc.load_gather(draft_v, [idx_c])
                tm = plsc.load_gather(tmax_v, [idx_c])
                matched = (dr == tm) | jnp.logical_not(valid)
                emit = (prefix > 0) & valid
                emitted = jnp.where(emit, tm, -1)
                plsc.store_scatter(out_local,
                                   [lanes, jnp.full((L,), p, jnp.int32)],
                                   emitted)
                prefix = prefix * jnp.where(matched, 1, 0)
                nm = nm + prefix * jnp.where(valid, 1, 0)
            plsc.store_scatter(out_local,
                               [lanes, jnp.full((L,), S, jnp.int32)],
                               jnp.full((L,), -1, jnp.int32))
            bv = jnp.where(nm == lengths, bonus_v[...], -1)
            lc = jnp.clip(lengths, 0, S)
            plsc.store_scatter(out_local, [lanes, lc], bv)
            nb_local[...] = nm
            pltpu.sync_copy(out_local, out_hbm.at[pl.ds(wid * L, L)])
            pltpu.sync_copy(nb_local, nb_hbm.at[pl.ds(wid * L, L)])

    return k(draft, tc_max, tc_idx, pm, pi, cu, bonus, params)


def kernel(draft_token_ids, num_spec_steps, cu_num_draft_tokens,
           target_logits, bonus_token_ids):
    B = cu_num_draft_tokens.shape[0]
    T = draft_token_ids.shape[0]
    S = T // B
    xT = jnp.swapaxes(target_logits, 0, 1)
    tc_max, tc_idx = _argmax_tc(xT, _VT_SPLIT)
    pm, pi = _argmax_sc_partial(xT)
    ns = jnp.asarray(num_spec_steps, jnp.int32)
    params = jnp.full((_SC_LANES,), ns - S, jnp.int32)
    out, num_bonus = _reject_sc(
        draft_token_ids.astype(jnp.int32), tc_max, tc_idx, pm, pi,
        cu_num_draft_tokens.astype(jnp.int32),
        bonus_token_ids.astype(jnp.int32), params)
    return out, num_bonus


# speedup vs baseline: 1.1768x; 1.1768x over previous
import functools

import jax
import jax.numpy as jnp
from jax import lax
from jax.experimental import pallas as pl
from jax.experimental.pallas import tpu as pltpu
from jax.experimental.pallas import tpu_sc as plsc

_TC_LANES = 128
_SC_LANES = 16


def _argmax_tc(logits):
    R, V = logits.shape
    xT = jnp.swapaxes(logits, 0, 1)
    C = 10240
    nb = (V + C - 1) // C
    big = jnp.iinfo(jnp.int32).max

    def body(x_ref, out_ref, vm_ref, vt_ref):
        j = pl.program_id(0)

        @pl.when(j == 0)
        def _init():
            vm_ref[...] = jnp.full((1, R), -jnp.inf, jnp.float32)
            vt_ref[...] = jnp.zeros((1, R), jnp.int32)

        def scan(valid_rows):
            blk = x_ref[...]
            iota0 = lax.broadcasted_iota(jnp.int32, (C, R), 0)
            if valid_rows is not None:
                blk = jnp.where(iota0 < valid_rows, blk, -jnp.inf)
            m = jnp.max(blk, axis=0, keepdims=True)
            im = jnp.min(jnp.where(blk == m, iota0, big), axis=0,
                         keepdims=True) + j * C
            upd = m > vm_ref[...]
            vm_ref[...] = jnp.where(upd, m, vm_ref[...])
            vt_ref[...] = jnp.where(upd, im, vt_ref[...])

        tail = V - (nb - 1) * C

        @pl.when(j < nb - 1)
        def _steady():
            scan(None)

        @pl.when(j == nb - 1)
        def _last():
            scan(None if tail == C else tail)
            out_ref[...] = vt_ref[...].reshape(R)

    out = pl.pallas_call(
        body,
        grid=(nb,),
        in_specs=[pl.BlockSpec((C, R), lambda j: (j, 0))],
        out_specs=pl.BlockSpec((R,), lambda j: (0,)),
        out_shape=jax.ShapeDtypeStruct((R,), jnp.int32),
        scratch_shapes=[
            pltpu.VMEM((1, R), jnp.float32),
            pltpu.VMEM((1, R), jnp.int32),
        ],
    )(xT)
    return out


def _reject_sc(draft, tmax, cu, bonus, params):
    B = cu.shape[0]
    T = draft.shape[0]
    S = T // B
    L = _SC_LANES
    NW = B // L
    W = S + 1
    mesh = plsc.VectorSubcoreMesh(core_axis_name="c", subcore_axis_name="s",
                                  num_cores=2, num_subcores=16)

    @functools.partial(
        pl.kernel,
        out_type=(jax.ShapeDtypeStruct((B, W), jnp.int32),
                  jax.ShapeDtypeStruct((B,), jnp.int32)),
        mesh=mesh,
        compiler_params=pltpu.CompilerParams(needs_layout_passes=False,
                                             skip_device_barrier=True),
        scratch_types=[
            pltpu.VMEM((B,), jnp.int32),
            pltpu.VMEM((T,), jnp.int32),
            pltpu.VMEM((T,), jnp.int32),
            pltpu.VMEM((L,), jnp.int32),
            pltpu.VMEM((L,), jnp.int32),
            pltpu.VMEM((L, W), jnp.int32),
            pltpu.VMEM((L,), jnp.int32),
        ],
    )
    def k(draft_hbm, tmax_hbm, cu_hbm, bonus_hbm, params_hbm, out_hbm, nb_hbm,
          cu_v, draft_v, tmax_v, bonus_v, params_v, out_local, nb_local):
        wid = lax.axis_index("s") * 2 + lax.axis_index("c")

        @pl.when(wid < NW)
        def _():
            pltpu.sync_copy(cu_hbm, cu_v)
            pltpu.sync_copy(draft_hbm, draft_v)
            pltpu.sync_copy(tmax_hbm, tmax_v)
            pltpu.sync_copy(bonus_hbm.at[pl.ds(wid * L, L)], bonus_v)
            pltpu.sync_copy(params_hbm, params_v)

            lanes = lax.broadcasted_iota(jnp.int32, (L,), 0)
            r = wid * L + lanes
            cu_r = plsc.load_gather(cu_v, [r])
            cu_rm1 = plsc.load_gather(cu_v, [jnp.maximum(r - 1, 0)])
            zero = jnp.zeros((L,), jnp.int32)
            starts = jnp.where(r == 0, zero, cu_rm1)
            lengths = cu_r - starts
            pos_off = params_v[...]

            prefix = jnp.ones((L,), jnp.int32)
            nm = zero
            for p in range(S):
                pos = pos_off + p
                idx = starts + pos
                valid = pos < lengths
                idx_c = jnp.clip(idx, 0, T - 1)
                dr = plsc.load_gather(draft_v, [idx_c])
                tm = plsc.load_gather(tmax_v, [idx_c])
                matched = (dr == tm) | jnp.logical_not(valid)
                emit = (prefix > 0) & valid
                emitted = jnp.where(emit, tm, -1)
                plsc.store_scatter(out_local,
                                   [lanes, jnp.full((L,), p, jnp.int32)],
                                   emitted)
                prefix = prefix * jnp.where(matched, 1, 0)
                nm = nm + prefix * jnp.where(valid, 1, 0)
            plsc.store_scatter(out_local,
                               [lanes, jnp.full((L,), S, jnp.int32)],
                               jnp.full((L,), -1, jnp.int32))
            bv = jnp.where(nm == lengths, bonus_v[...], -1)
            lc = jnp.clip(lengths, 0, S)
            plsc.store_scatter(out_local, [lanes, lc], bv)
            nb_local[...] = nm
            pltpu.sync_copy(out_local, out_hbm.at[pl.ds(wid * L, L)])
            pltpu.sync_copy(nb_local, nb_hbm.at[pl.ds(wid * L, L)])

    return k(draft, tmax, cu, bonus, params)


def kernel(draft_token_ids, num_spec_steps, cu_num_draft_tokens,
           target_logits, bonus_token_ids):
    B = cu_num_draft_tokens.shape[0]
    T = draft_token_ids.shape[0]
    S = T // B
    tmax = _argmax_tc(target_logits)
    ns = jnp.asarray(num_spec_steps, jnp.int32)
    params = jnp.full((_SC_LANES,), ns - S, jnp.int32)
    out, num_bonus = _reject_sc(
        draft_token_ids.astype(jnp.int32), tmax,
        cu_num_draft_tokens.astype(jnp.int32),
        bonus_token_ids.astype(jnp.int32), params)
    return out, num_bonus
